# baseline (speedup 1.0000x reference)
import jax
import jax.numpy as jnp
from jax.experimental import pallas as pl
from jax.experimental.pallas import tpu as pltpu

_THRESHOLD = 25.0
_BM = 512


def _body(x_ref, W1_ref, b1_ref, Wc_ref, bc_ref, W2_ref, b2_ref, Wf_ref,
          bf_ref, out_ref):
    x = x_ref[...]
    h = jnp.maximum(
        jnp.dot(x, W1_ref[...], preferred_element_type=jnp.float32)
        + b1_ref[...], 0.0)
    cache_pred = (jnp.dot(h, Wc_ref[...], preferred_element_type=jnp.float32)
                  + bc_ref[...])
    mx = jnp.max(jnp.exp(cache_pred), axis=1, keepdims=True)
    h2 = jnp.maximum(
        jnp.dot(h, W2_ref[...], preferred_element_type=jnp.float32)
        + b2_ref[...], 0.0)
    final_out = (jnp.dot(h2, Wf_ref[...], preferred_element_type=jnp.float32)
                 + bf_ref[...])
    out_ref[...] = jnp.where(mx > _THRESHOLD, cache_pred, final_out)


def kernel(x, W1, b1, Wc, bc, W2, b2, Wf, bf):
    B, D = x.shape
    NC = Wc.shape[1]

    def _full(shape):
        return pl.BlockSpec(shape, lambda i: (0, 0))

    return pl.pallas_call(
        _body,
        grid=(B // _BM,),
        in_specs=[
            pl.BlockSpec((_BM, D), lambda i: (i, 0)),
            _full((D, D)),
            _full((1, D)),
            _full((D, NC)),
            _full((1, NC)),
            _full((D, D)),
            _full((1, D)),
            _full((D, NC)),
            _full((1, NC)),
        ],
        out_specs=pl.BlockSpec((_BM, NC), lambda i: (i, 0)),
        out_shape=jax.ShapeDtypeStruct((B, NC), jnp.float32),
        compiler_params=pltpu.CompilerParams(
            dimension_semantics=("parallel",)),
    )(x, W1, b1.reshape(1, D), Wc, bc.reshape(1, NC), W2,
      b2.reshape(1, D), Wf, bf.reshape(1, NC))
